# Initial kernel scaffold; baseline (speedup 1.0000x reference)
import jax
import jax.numpy as jnp
from jax.experimental import pallas as pl


def kernel(positions, pe):
    raise NotImplementedError("write your pallas kernel here")


# speedup vs baseline: 6.7647x; 6.7647x over previous
import jax
import jax.numpy as jnp
from jax.experimental import pallas as pl
from jax.experimental.pallas import tpu as pltpu
from jax.experimental.pallas import tpu_sc as plsc

DIM = 128
MAX_LEN = 2048
LANES = 16
WINDOW = 128


def _sc_gather(idx_flat, pe):
    B = idx_flat.shape[1]
    mesh = plsc.VectorSubcoreMesh(core_axis_name="core", subcore_axis_name="subcore")

    @pl.kernel(
        out_type=jax.ShapeDtypeStruct((B, DIM), pe.dtype),
        mesh=mesh,
        scratch_types=[pltpu.VMEM((WINDOW,), jnp.int32)],
    )
    def k(pe_hbm, i_hbm, o_hbm, idx_v):
        def body(i_vmem, o_vmem):
            row = i_vmem.at[0]

            @pl.loop(0, WINDOW, step=LANES)
            def _(c):
                raw = row.at[pl.ds(c, LANES)][...]
                idx_v.at[pl.ds(c, LANES)][...] = jnp.minimum(
                    jnp.maximum(raw, 0), MAX_LEN - 1
                )

            pltpu.sync_copy(pe_hbm.at[idx_v], o_vmem)

        pltpu.emit_pipeline(
            body,
            grid=(B // WINDOW,),
            in_specs=[pl.BlockSpec((1, WINDOW), lambda i: (0, i))],
            out_specs=[pl.BlockSpec((WINDOW, DIM), lambda i: (i, 0))],
            core_axis_name=("core", "subcore"),
            dimension_semantics=(pltpu.PARALLEL,),
        )(i_hbm, o_hbm)

    return k(pe, idx_flat)


@jax.jit
def kernel(positions, pe):
    b0, b1 = positions.shape
    idx_flat = positions.reshape(1, b0 * b1)
    out = _sc_gather(idx_flat, pe)
    return out.reshape(b0, b1, DIM)


# speedup vs baseline: 14.3397x; 2.1198x over previous
import jax
import jax.numpy as jnp
from jax import lax
from jax.experimental import pallas as pl
from jax.experimental.pallas import tpu as pltpu
from jax.experimental.pallas import tpu_sc as plsc

DIM = 128
MAX_LEN = 2048
LANES = 16
WINDOW = 128


def _sc_gather(idx_flat, pe):
    B = idx_flat.shape[1]
    mesh = plsc.VectorSubcoreMesh(core_axis_name="core", subcore_axis_name="subcore")

    @pl.kernel(
        out_type=jax.ShapeDtypeStruct((B, DIM), pe.dtype),
        mesh=mesh,
        scratch_types=[
            pltpu.VMEM((WINDOW,), jnp.int32),
            pltpu.VMEM_SHARED((MAX_LEN, DIM), pe.dtype),
        ],
    )
    def k(pe_hbm, i_hbm, o_hbm, idx_v, pe_sh):
        @pl.when(lax.axis_index("subcore") == 0)
        def _():
            pltpu.sync_copy(pe_hbm, pe_sh)

        plsc.subcore_barrier()

        def body(i_vmem, o_vmem):
            row = i_vmem.at[0]

            @pl.loop(0, WINDOW, step=LANES)
            def _(c):
                raw = row.at[pl.ds(c, LANES)][...]
                idx_v.at[pl.ds(c, LANES)][...] = jnp.minimum(
                    jnp.maximum(raw, 0), MAX_LEN - 1
                )

            pltpu.sync_copy(pe_sh.at[idx_v], o_vmem)

        pltpu.emit_pipeline(
            body,
            grid=(B // WINDOW,),
            in_specs=[pl.BlockSpec((1, WINDOW), lambda i: (0, i))],
            out_specs=[pl.BlockSpec((WINDOW, DIM), lambda i: (i, 0))],
            core_axis_name=("core", "subcore"),
            dimension_semantics=(pltpu.PARALLEL,),
        )(i_hbm, o_hbm)

    return k(pe, idx_flat)


@jax.jit
def kernel(positions, pe):
    b0, b1 = positions.shape
    idx_flat = positions.reshape(1, b0 * b1)
    out = _sc_gather(idx_flat, pe)
    return out.reshape(b0, b1, DIM)
